# baseline (speedup 1.0000x reference)
import jax
import jax.numpy as jnp
from jax.experimental import pallas as pl
from jax.experimental.pallas import tpu as pltpu

N = 4096
BR = 128
NB = N // BR
K = 2048
D_IN = 128
D_HID = 64
D_OUT = 40


def _topk_gate(s_col):
    s_wide = s_col.reshape(32, 128)
    s = s_wide + 0.0
    b = jax.lax.bitcast_convert_type(s, jnp.int32)
    imin = jnp.int32(-2147483648)
    key = jnp.where(b >= 0, b, imin - b)

    def tstep(j, t):
        q = t + (jnp.int32(1) << (jnp.int32(30) - j))
        cnt = jnp.sum(jnp.where(key >= q, 1, 0).astype(jnp.int32))
        return jnp.where(cnt >= K, q, t)

    t = jax.lax.fori_loop(0, 31, tstep, imin)

    eq = key == t
    rem = K - jnp.sum(jnp.where(key > t, 1, 0).astype(jnp.int32))
    ri = jax.lax.broadcasted_iota(jnp.int32, s.shape, 0)
    ci = jax.lax.broadcasted_iota(jnp.int32, s.shape, 1)
    idx = ri * s.shape[1] + ci

    def mstep(j, m):
        q = m + (jnp.int32(1) << (jnp.int32(12) - j))
        cnt = jnp.sum(jnp.where(eq & (idx < q), 1, 0).astype(jnp.int32))
        return jnp.where(cnt <= rem, q, m)

    mm = jax.lax.fori_loop(0, 13, mstep, jnp.int32(0))

    bc = jax.lax.bitcast_convert_type(s_col + 0.0, jnp.int32)
    key_c = jnp.where(bc >= 0, bc, imin - bc)
    ic = jax.lax.broadcasted_iota(jnp.int32, s_col.shape, 0)
    mask_c = (key_c > t) | ((key_c == t) & (ic < mm))
    return jnp.where(mask_c, jax.nn.sigmoid(s_col), jnp.float32(0.0))


def _body(h_ref, lw_ref, w1_ref, wp_ref, p_ref, wu_ref, w2_ref, a_ref,
          out_ref,
          ab_scr, dinv_scr, x_scr, z_scr,
          h1_scr, hp_scr, s_scr):
    ph = pl.program_id(0)
    i = pl.program_id(1)
    rs = pl.ds(i * BR, BR)

    @pl.when(ph == 0)
    def _p0():
        a = a_ref[...]
        dinv_scr[rs, :] = jnp.sum(a, axis=1, keepdims=True)
        ab_scr[rs, :] = a.astype(jnp.bfloat16)

    @pl.when((ph == 1) & (i == 0))
    def _pro1():
        dg = dinv_scr[...]
        dinv_scr[...] = jnp.where(dg > 0.0, jax.lax.rsqrt(dg), 0.0)
        x = jnp.dot(h_ref[...], w1_ref[...],
                    preferred_element_type=jnp.float32)
        x_scr[...] = x
        z_scr[...] = (x * dinv_scr[...]).astype(jnp.bfloat16)

    @pl.when((ph == 2) & (i == 0))
    def _pro2():
        x = jnp.dot(h1_scr[...], wp_ref[...],
                    preferred_element_type=jnp.float32)
        x_scr[...] = x
        z_scr[...] = (x * dinv_scr[...]).astype(jnp.bfloat16)

    @pl.when((ph == 3) & (i == 0))
    def _pro3():
        h2 = h1_scr[...] + s_scr[...] * hp_scr[...]
        x = jnp.dot(h2, wu_ref[...], preferred_element_type=jnp.float32)
        x_scr[...] = x
        z_scr[...] = (x * dinv_scr[...]).astype(jnp.bfloat16)

    @pl.when((ph == 4) & (i == 0))
    def _pro4():
        x = jnp.dot(h1_scr[...], w2_ref[...],
                    preferred_element_type=jnp.float32)
        x_scr[...] = x
        z_scr[...] = (x * dinv_scr[...]).astype(jnp.bfloat16)

    @pl.when(ph > 0)
    def _gcn():
        dv = dinv_scr[rs, :]
        lw = lw_ref[...]
        ab = ab_scr[rs, :]

        @pl.when(ph == 1)
        def _g1():
            acc = jnp.dot(ab, z_scr[...], preferred_element_type=jnp.float32)
            h1_scr[rs, :] = jnp.maximum(dv * acc + lw * x_scr[rs, :], 0.0)

        @pl.when(ph == 2)
        def _g2():
            acc = jnp.dot(ab, z_scr[...], preferred_element_type=jnp.float32)
            hp = jnp.maximum(dv * acc + lw * x_scr[rs, :], 0.0)
            hp_scr[rs, :] = hp
            pv = p_ref[...]
            pn = jnp.sqrt(jnp.sum(pv * pv)) + 1e-12
            s_scr[rs, :] = jnp.dot(hp, pv,
                                   preferred_element_type=jnp.float32) / pn

            @pl.when(i == NB - 1)
            def _mask():
                s_scr[...] = _topk_gate(s_scr[...])

        @pl.when(ph == 3)
        def _g3():
            acc = jnp.dot(ab, z_scr[...], preferred_element_type=jnp.float32)
            h1_scr[rs, :] = jnp.maximum(dv * acc + lw * x_scr[rs, :], 0.0)

        @pl.when(ph == 4)
        def _g4():
            acc = jnp.dot(ab, z_scr[...], preferred_element_type=jnp.float32)
            h = jnp.maximum(dv * acc + lw * x_scr[rs, :], 0.0)
            hh = h[:, :D_OUT]
            m = jnp.max(hh, axis=1, keepdims=True)
            e = jnp.exp(hh - m)
            lse = jnp.log(jnp.sum(e, axis=1, keepdims=True)) + m
            out_ref[...] = hh - lse


def kernel(H, A, loop_w, W1, Wp, p, Wu, W2):
    lw = loop_w.reshape(N, 1)
    p2 = p.reshape(D_HID, 1)
    W2p = jnp.pad(W2, ((0, 0), (0, D_HID - D_OUT)))

    def _full(shape):
        return pl.BlockSpec(shape, lambda ph, i: (0, 0))

    out = pl.pallas_call(
        _body,
        grid=(5, NB),
        in_specs=[
            _full((N, D_IN)),
            pl.BlockSpec((BR, 1), lambda ph, i: (i, 0)),
            _full((D_IN, D_HID)),
            _full((D_HID, D_HID)),
            _full((D_HID, 1)),
            _full((D_HID, D_HID)),
            _full((D_HID, D_HID)),
            pl.BlockSpec((BR, N),
                         lambda ph, i: (jnp.where(ph == 0, i, NB - 1), 0)),
        ],
        out_specs=pl.BlockSpec((BR, D_OUT),
                               lambda ph, i: (jnp.where(ph == 4, i, 0), 0)),
        out_shape=jax.ShapeDtypeStruct((N, D_OUT), jnp.float32),
        scratch_shapes=[
            pltpu.VMEM((N, N), jnp.bfloat16),
            pltpu.VMEM((N, 1), jnp.float32),
            pltpu.VMEM((N, D_HID), jnp.float32),
            pltpu.VMEM((N, D_HID), jnp.bfloat16),
            pltpu.VMEM((N, D_HID), jnp.float32),
            pltpu.VMEM((N, D_HID), jnp.float32),
            pltpu.VMEM((N, 1), jnp.float32),
        ],
    )(H, lw, W1, Wp, p2, Wu, W2p, A)
    return out


# speedup vs baseline: 1.3846x; 1.3846x over previous
import jax
import jax.numpy as jnp
from jax.experimental import pallas as pl
from jax.experimental.pallas import tpu as pltpu

N = 4096
BR = 128
NS = N // BR
CR = 512
NC = N // CR
K = 2048
D_IN = 128
D_HID = 64
D_OUT = 40


def _topk_gate(s_col):
    s_wide = s_col.reshape(32, 128)
    s = s_wide + 0.0
    b = jax.lax.bitcast_convert_type(s, jnp.int32)
    imin = jnp.int32(-2147483648)
    key = jnp.where(b >= 0, b, imin - b)

    def tstep(j, t):
        q = t + (jnp.int32(1) << (jnp.int32(30) - j))
        cnt = jnp.sum(jnp.where(key >= q, 1, 0).astype(jnp.int32))
        return jnp.where(cnt >= K, q, t)

    t = jax.lax.fori_loop(0, 31, tstep, imin)

    eq = key == t
    rem = K - jnp.sum(jnp.where(key > t, 1, 0).astype(jnp.int32))
    ri = jax.lax.broadcasted_iota(jnp.int32, s.shape, 0)
    ci = jax.lax.broadcasted_iota(jnp.int32, s.shape, 1)
    idx = ri * s.shape[1] + ci

    def mstep(j, m):
        q = m + (jnp.int32(1) << (jnp.int32(12) - j))
        cnt = jnp.sum(jnp.where(eq & (idx < q), 1, 0).astype(jnp.int32))
        return jnp.where(cnt <= rem, q, m)

    mm = jax.lax.fori_loop(0, 13, mstep, jnp.int32(0))

    bc = jax.lax.bitcast_convert_type(s_col + 0.0, jnp.int32)
    key_c = jnp.where(bc >= 0, bc, imin - bc)
    ic = jax.lax.broadcasted_iota(jnp.int32, s_col.shape, 0)
    mask_c = (key_c > t) | ((key_c == t) & (ic < mm))
    return jnp.where(mask_c, jax.nn.sigmoid(s_col), jnp.float32(0.0))


def _body(h_ref, lw_ref, w1_ref, wp_ref, p_ref, wu_ref, w2_ref, a_ref,
          out_ref,
          ab_scr, dinv_scr, x_scr, z_scr,
          h1_scr, hp_scr, s_scr):
    t = pl.program_id(0)

    @pl.when(t < NS)
    def _p0():
        a = a_ref[...]
        rs0 = pl.ds(t * BR, BR)
        dinv_scr[rs0, :] = jnp.sum(a, axis=1, keepdims=True)
        ab_scr[rs0, :] = a.astype(jnp.bfloat16)

    @pl.when(t == NS)
    def _pro1():
        dg = dinv_scr[...]
        dinv_scr[...] = jnp.where(dg > 0.0, jax.lax.rsqrt(dg), 0.0)
        x = jnp.dot(h_ref[...], w1_ref[...],
                    preferred_element_type=jnp.float32)
        x_scr[...] = x
        z_scr[...] = (x * dinv_scr[...]).astype(jnp.bfloat16)

    @pl.when(t == NS + NC)
    def _pro2():
        x = jnp.dot(h1_scr[...], wp_ref[...],
                    preferred_element_type=jnp.float32)
        x_scr[...] = x
        z_scr[...] = (x * dinv_scr[...]).astype(jnp.bfloat16)

    @pl.when(t == NS + 2 * NC)
    def _pro3():
        h2 = h1_scr[...] + s_scr[...] * hp_scr[...]
        x = jnp.dot(h2, wu_ref[...], preferred_element_type=jnp.float32)
        x_scr[...] = x
        z_scr[...] = (x * dinv_scr[...]).astype(jnp.bfloat16)

    @pl.when(t == NS + 3 * NC)
    def _pro4():
        x = jnp.dot(h1_scr[...], w2_ref[...],
                    preferred_element_type=jnp.float32)
        x_scr[...] = x
        z_scr[...] = (x * dinv_scr[...]).astype(jnp.bfloat16)

    @pl.when(t >= NS)
    def _gcn():
        tt = jnp.maximum(t - NS, 0)
        p = tt // NC
        rs = pl.ds((tt % NC) * CR, CR)
        dv = dinv_scr[rs, :]
        lw = lw_ref[...]
        ab = ab_scr[rs, :]
        acc = jnp.dot(ab, z_scr[...], preferred_element_type=jnp.float32)
        h = jnp.maximum(dv * acc + lw * x_scr[rs, :], 0.0)

        @pl.when(p == 0)
        def _g1():
            h1_scr[rs, :] = h

        @pl.when(p == 1)
        def _g2():
            hp_scr[rs, :] = h
            pv = p_ref[...]
            pn = jnp.sqrt(jnp.sum(pv * pv)) + 1e-12
            s_scr[rs, :] = jnp.dot(h, pv,
                                   preferred_element_type=jnp.float32) / pn

            @pl.when(tt == 2 * NC - 1)
            def _mask():
                s_scr[...] = _topk_gate(s_scr[...])

        @pl.when(p == 2)
        def _g3():
            h1_scr[rs, :] = h

        @pl.when(p == 3)
        def _g4():
            hh = h[:, :D_OUT]
            m = jnp.max(hh, axis=1, keepdims=True)
            e = jnp.exp(hh - m)
            lse = jnp.log(jnp.sum(e, axis=1, keepdims=True)) + m
            out_ref[...] = hh - lse


def kernel(H, A, loop_w, W1, Wp, p, Wu, W2):
    lw = loop_w.reshape(N, 1)
    p2 = p.reshape(D_HID, 1)
    W2p = jnp.pad(W2, ((0, 0), (0, D_HID - D_OUT)))

    def _full(shape):
        return pl.BlockSpec(shape, lambda t: (0, 0))

    out = pl.pallas_call(
        _body,
        grid=(NS + 4 * NC,),
        in_specs=[
            _full((N, D_IN)),
            pl.BlockSpec((CR, 1),
                         lambda t: ((jnp.maximum(t, NS) - NS) % NC, 0)),
            _full((D_IN, D_HID)),
            _full((D_HID, D_HID)),
            _full((D_HID, 1)),
            _full((D_HID, D_HID)),
            _full((D_HID, D_HID)),
            pl.BlockSpec((BR, N),
                         lambda t: (jnp.minimum(t, NS - 1), 0)),
        ],
        out_specs=pl.BlockSpec(
            (CR, D_OUT),
            lambda t: (jnp.where(t >= NS + 3 * NC, t - (NS + 3 * NC), 0), 0)),
        out_shape=jax.ShapeDtypeStruct((N, D_OUT), jnp.float32),
        scratch_shapes=[
            pltpu.VMEM((N, N), jnp.bfloat16),
            pltpu.VMEM((N, 1), jnp.float32),
            pltpu.VMEM((N, D_HID), jnp.float32),
            pltpu.VMEM((N, D_HID), jnp.bfloat16),
            pltpu.VMEM((N, D_HID), jnp.float32),
            pltpu.VMEM((N, D_HID), jnp.float32),
            pltpu.VMEM((N, 1), jnp.float32),
        ],
    )(H, lw, W1, Wp, p2, Wu, W2p, A)
    return out
